# baseline (speedup 1.0000x reference)
import jax
import jax.numpy as jnp
from jax import lax
from jax.experimental import pallas as pl
from jax.experimental.pallas import tpu as pltpu
from jax.experimental.pallas import tpu_sc as plsc

SEQ = 200
DIM = 32
NUM_CORES = 2
NUM_SUBCORES = 16
NUM_WORKERS = NUM_CORES * NUM_SUBCORES
BCHUNK = 128
NBUF = 4


def _pe_table():
    pos = jnp.arange(SEQ, dtype=jnp.float32)[:, None]
    j = jnp.arange(DIM, dtype=jnp.float32)[None, :]
    angle = pos / (10000.0 ** (j / float(DIM)))
    even = (jnp.arange(DIM)[None, :] % 2) == 0
    return jnp.where(even, jnp.sin(angle), jnp.cos(angle)).astype(jnp.float32)


def _sc_body(x_hbm, pe_hbm, table_hbm, out_hbm, idx_v, pe_v, rows_g,
             *rest):
    rows_o = rest[:NBUF]
    sem_g = rest[NBUF:2 * NBUF]
    sem_s = rest[2 * NBUF:]
    wid = lax.axis_index("s") * NUM_CORES + lax.axis_index("c")
    n_outer = SEQ // NBUF
    i16 = lax.iota(jnp.int32, 16)
    tr_lo = i16 // 8
    tr_hi = tr_lo + 2
    r_j = i16 % 8

    pltpu.sync_copy(x_hbm.at[pl.ds(0, SEQ // 8), wid], idx_v)
    pltpu.sync_copy(pe_hbm, pe_v)

    def gather_copy(s, b):
        return pltpu.make_async_copy(
            table_hbm.at[idx_v.at[s // 8, s % 8]], rows_g.at[b], sem_g[b])

    def store_copies(s, b):
        return [
            pltpu.make_async_copy(rows_o[b].at[tr], out_hbm.at[s, tr, wid],
                                  sem_s[b])
            for tr in range(4)
        ]

    for b in range(NBUF):
        gather_copy(b, b).start()

    def outer_body(k, carry):
        for b in range(NBUF):
            s = k * NBUF + b
            gather_copy(s, b).wait()

            @pl.when(k > 0)
            def _wait_store():
                for c in store_copies(s, b):
                    c.wait()

            def col_body(c, c_vec):
                lo = rows_g[b, c, 0:16] + pe_v[s, 0:16]
                hi = rows_g[b, c, 16:32] + pe_v[s, 16:32]
                plsc.store_scatter(rows_o[b], [tr_lo, r_j, c_vec], lo)
                plsc.store_scatter(rows_o[b], [tr_hi, r_j, c_vec], hi)
                return c_vec + 1

            lax.fori_loop(0, BCHUNK, col_body, i16 * 0, unroll=8)

            for c in store_copies(s, b):
                c.start()

            @pl.when(k < n_outer - 1)
            def _next_gather():
                gather_copy(s + NBUF, b).start()
        return carry

    lax.fori_loop(0, n_outer, outer_body, 0)

    for b in range(NBUF):
        s = (n_outer - 1) * NBUF + b
        for c in store_copies(s, b):
            c.wait()


@jax.jit
def kernel(x, table):
    batch, seq = x.shape
    pe = _pe_table()
    x5d = (x.T.reshape(seq // 8, 8, batch // 128, 128)
           .transpose(0, 2, 1, 3))
    mesh = plsc.VectorSubcoreMesh(
        core_axis_name="c", subcore_axis_name="s",
        num_cores=NUM_CORES, num_subcores=NUM_SUBCORES)
    run = pl.kernel(
        _sc_body,
        out_type=jax.ShapeDtypeStruct((seq, DIM // 8, batch // 128, 8, 128),
                                      jnp.float32),
        mesh=mesh,
        scratch_types=[
            pltpu.VMEM((seq // 8, 8, BCHUNK), jnp.int32),
            pltpu.VMEM((SEQ, DIM), jnp.float32),
            pltpu.VMEM((NBUF, BCHUNK, DIM), jnp.float32),
        ] + [pltpu.VMEM((DIM // 8, 8, BCHUNK), jnp.float32)] * NBUF
          + [pltpu.SemaphoreType.DMA] * (2 * NBUF),
        compiler_params=pltpu.CompilerParams(use_tc_tiling_on_sc=False,
                                             needs_layout_passes=False),
    )
    out5d = run(x5d, pe, table)
    return out5d.transpose(2, 4, 0, 1, 3).reshape(batch, seq, DIM)


# speedup vs baseline: 1.5488x; 1.5488x over previous
import jax
import jax.numpy as jnp
from jax import lax
from jax.experimental import pallas as pl
from jax.experimental.pallas import tpu as pltpu
from jax.experimental.pallas import tpu_sc as plsc

SEQ = 200
DIM = 32
NUM_CORES = 2
NUM_SUBCORES = 16
NUM_WORKERS = NUM_CORES * NUM_SUBCORES
BCHUNK = 128
NBUF = 4


def _pe_table():
    pos = jnp.arange(SEQ, dtype=jnp.float32)[:, None]
    j = jnp.arange(DIM, dtype=jnp.float32)[None, :]
    angle = pos / (10000.0 ** (j / float(DIM)))
    even = (jnp.arange(DIM)[None, :] % 2) == 0
    return jnp.where(even, jnp.sin(angle), jnp.cos(angle)).astype(jnp.float32)


def _sc_body(x_hbm, pe_hbm, table_hbm, out_hbm, idx_v, pe_v, rows_g,
             *rest):
    rows_o = rest[:NBUF]
    sem_g = rest[NBUF:2 * NBUF]
    sem_s = rest[2 * NBUF:]
    wid = lax.axis_index("s") * NUM_CORES + lax.axis_index("c")
    n_outer = SEQ // NBUF
    i16 = lax.iota(jnp.int32, 16)
    tr_lo = i16 // 8
    tr_hi = tr_lo + 2
    r_j = i16 % 8

    pltpu.sync_copy(x_hbm.at[pl.ds(0, SEQ // 8), wid], idx_v)
    pltpu.sync_copy(pe_hbm, pe_v)

    def gather_copy(s, b):
        return pltpu.make_async_copy(
            table_hbm.at[idx_v.at[s // 8, s % 8]], rows_g.at[b], sem_g[b])

    def store_copies(s, b):
        return [
            pltpu.make_async_copy(
                rows_o[b].at[tr, pl.ds(0, 8), pl.ds(0, BCHUNK)],
                out_hbm.at[s, tr, wid], sem_s[b])
            for tr in range(4)
        ]

    for b in range(NBUF):
        gather_copy(b, b).start()

    def outer_body(k, carry):
        for b in range(NBUF):
            s = k * NBUF + b
            gather_copy(s, b).wait()

            @pl.when(k > 0)
            def _wait_store():
                for c in store_copies(s, b):
                    c.wait()

            def col_body(c, c_vec):
                lo = rows_g[b, c, 0:16] + pe_v[s, 0:16]
                hi = rows_g[b, c, 16:32] + pe_v[s, 16:32]
                plsc.store_scatter(rows_o[b], [tr_lo, r_j, c_vec], lo)
                plsc.store_scatter(rows_o[b], [tr_hi, r_j, c_vec], hi)
                return c_vec + 1

            lax.fori_loop(0, BCHUNK, col_body, i16 * 0, unroll=8)

            for c in store_copies(s, b):
                c.start()

            @pl.when(k < n_outer - 1)
            def _next_gather():
                gather_copy(s + NBUF, b).start()
        return carry

    lax.fori_loop(0, n_outer, outer_body, 0)

    for b in range(NBUF):
        s = (n_outer - 1) * NBUF + b
        for c in store_copies(s, b):
            c.wait()


@jax.jit
def kernel(x, table):
    batch, seq = x.shape
    pe = _pe_table()
    x5d = (x.T.reshape(seq // 8, 8, batch // 128, 128)
           .transpose(0, 2, 1, 3))
    mesh = plsc.VectorSubcoreMesh(
        core_axis_name="c", subcore_axis_name="s",
        num_cores=NUM_CORES, num_subcores=NUM_SUBCORES)
    run = pl.kernel(
        _sc_body,
        out_type=jax.ShapeDtypeStruct((seq, DIM // 8, batch // 128, 8, 128),
                                      jnp.float32),
        mesh=mesh,
        scratch_types=[
            pltpu.VMEM((seq // 8, 8, BCHUNK), jnp.int32),
            pltpu.VMEM((SEQ, DIM), jnp.float32),
            pltpu.VMEM((NBUF, BCHUNK, DIM), jnp.float32),
        ] + [pltpu.VMEM((DIM // 8, 8, BCHUNK + 1), jnp.float32)] * NBUF
          + [pltpu.SemaphoreType.DMA] * (2 * NBUF),
        compiler_params=pltpu.CompilerParams(use_tc_tiling_on_sc=False,
                                             needs_layout_passes=False),
    )
    out5d = run(x5d, pe, table)
    return out5d.transpose(2, 4, 0, 1, 3).reshape(batch, seq, DIM)
